# baseline (speedup 1.0000x reference)
import functools

import jax
import jax.numpy as jnp
from jax import lax
from jax.experimental import pallas as pl
from jax.experimental.pallas import tpu as pltpu
from jax.experimental.pallas import tpu_sc as plsc

N_TOK = 4096
M = 2048
HIDDEN = 512
NUM_EXPERTS = 8
TB = 256
N_TB = N_TOK // TB
GB = 256
P_PAD = 2 * N_TOK + NUM_EXPERTS * GB
NBLK = P_PAD // GB
NW = 32
TPW = N_TOK // NW
NCH = TPW // 16


def _router_kernel(xb_ref, rw_ref, tri_ref, e1_ref, e2_ref, r1_ref, r2_ref,
                   w1_ref, w2_ref, cnt_ref, carry_ref):
    i = pl.program_id(0)

    @pl.when(i == 0)
    def _():
        carry_ref[...] = jnp.zeros_like(carry_ref)

    xb = xb_ref[...]
    logits = lax.dot_general(xb, rw_ref[...], (((1,), (1,)), ((), ())),
                             preferred_element_type=jnp.float32)
    m1 = jnp.max(logits, axis=1, keepdims=True)
    cols = lax.broadcasted_iota(jnp.int32, logits.shape, 1)
    idx1 = jnp.min(jnp.where(logits == m1, cols, NUM_EXPERTS),
                   axis=1, keepdims=True)
    masked = jnp.where(cols == idx1, -jnp.inf, logits)
    m2 = jnp.max(masked, axis=1, keepdims=True)
    idx2 = jnp.min(jnp.where(masked == m2, cols, NUM_EXPERTS),
                   axis=1, keepdims=True)
    ex = jnp.exp(m2 - m1)
    den = 1.0 + ex
    w1v = 1.0 / den
    w2v = ex / den

    oh1 = (cols == idx1).astype(jnp.bfloat16)
    oh2 = (cols == idx2).astype(jnp.bfloat16)
    tri = tri_ref[...]
    pre1 = lax.dot_general(tri, oh1, (((1,), (0,)), ((), ())),
                           preferred_element_type=jnp.float32)
    pre2 = lax.dot_general(tri, oh2, (((1,), (0,)), ((), ())),
                           preferred_element_type=jnp.float32)
    sum1 = jnp.sum(oh1.astype(jnp.float32), axis=0, keepdims=True)
    sum2 = jnp.sum(oh2.astype(jnp.float32), axis=0, keepdims=True)
    carry = carry_ref[...]
    rank1 = jnp.sum(jnp.where(cols == idx1, pre1 + carry, 0.0),
                    axis=1, keepdims=True)
    rank2 = jnp.sum(jnp.where(cols == idx2, pre2 + sum1 + carry, 0.0),
                    axis=1, keepdims=True)
    new_carry = carry + sum1 + sum2
    carry_ref[...] = new_carry
    cnt_ref[...] = new_carry.astype(jnp.int32)

    e1_ref[...] = idx1.astype(jnp.int32).reshape(1, TB, 1)
    e2_ref[...] = idx2.astype(jnp.int32).reshape(1, TB, 1)
    r1_ref[...] = rank1.astype(jnp.int32).reshape(1, TB, 1)
    r2_ref[...] = rank2.astype(jnp.int32).reshape(1, TB, 1)
    w1_ref[...] = w1v.reshape(1, TB, 1)
    w2_ref[...] = w2v.reshape(1, TB, 1)


def _run_router(xb, rwb):
    tri = jnp.tril(jnp.ones((TB, TB), jnp.bfloat16), -1)
    vec = jax.ShapeDtypeStruct((N_TB, TB, 1), jnp.int32)
    vecf = jax.ShapeDtypeStruct((N_TB, TB, 1), jnp.float32)
    blk = pl.BlockSpec((1, TB, 1), lambda i: (i, 0, 0))
    return pl.pallas_call(
        _router_kernel,
        grid=(N_TB,),
        in_specs=[
            pl.BlockSpec((TB, M), lambda i: (i, 0)),
            pl.BlockSpec((NUM_EXPERTS, M), lambda i: (0, 0)),
            pl.BlockSpec((TB, TB), lambda i: (0, 0)),
        ],
        out_specs=[blk, blk, blk, blk, blk, blk,
                   pl.BlockSpec((1, NUM_EXPERTS), lambda i: (0, 0))],
        out_shape=[vec, vec, vec, vec, vecf, vecf,
                   jax.ShapeDtypeStruct((1, NUM_EXPERTS), jnp.int32)],
        scratch_shapes=[pltpu.VMEM((1, NUM_EXPERTS), jnp.float32)],
    )(xb, rwb, tri)


CHB = 16
NCHB = TPW // CHB


def _dispatch_body(x_hbm, d1_hbm, d2_hbm, xg_hbm,
                   d1_v, d2_v, xrow_v, lsem, ssem):
    c = lax.axis_index("c")
    s = lax.axis_index("s")
    wid = s * 2 + c
    base = wid * TPW
    pltpu.sync_copy(d1_hbm.at[wid], d1_v)
    pltpu.sync_copy(d2_hbm.at[wid], d2_v)
    loads = [None] * NCHB
    scats = [None] * NCHB
    for j in range(min(2, NCHB)):
        loads[j] = pltpu.async_copy(
            x_hbm.at[pl.ds(base + j * CHB, CHB)], xrow_v.at[j % 2], lsem)
    for j in range(NCHB):
        b = j % 2
        loads[j].wait()
        i1 = d1_v[j]
        i2 = d2_v[j]
        scats[j] = (pltpu.async_copy(xrow_v.at[b], xg_hbm.at[i1], ssem),
                    pltpu.async_copy(xrow_v.at[b], xg_hbm.at[i2], ssem))
        nxt = j + 2
        if nxt < NCHB:
            for cd in scats[nxt - 2]:
                cd.wait()
            loads[nxt] = pltpu.async_copy(
                x_hbm.at[pl.ds(base + nxt * CHB, CHB)], xrow_v.at[b], lsem)
    for j in range(max(0, NCHB - 2), NCHB):
        for cd in scats[j]:
            cd.wait()


def _run_dispatch(x, d1, d2):
    mesh = plsc.VectorSubcoreMesh(core_axis_name="c", subcore_axis_name="s",
                                   num_cores=2, num_subcores=16)
    fn = pl.kernel(
        _dispatch_body,
        out_type=jax.ShapeDtypeStruct((P_PAD, M), jnp.float32),
        mesh=mesh,
        compiler_params=pltpu.CompilerParams(needs_layout_passes=False),
        scratch_types=[
            pltpu.VMEM((NCHB, CHB), jnp.int32),
            pltpu.VMEM((NCHB, CHB), jnp.int32),
            pltpu.VMEM((2, CHB, M), jnp.float32),
            pltpu.SemaphoreType.DMA,
            pltpu.SemaphoreType.DMA,
        ],
    )
    return fn(x, d1.reshape(NW, NCHB, CHB), d2.reshape(NW, NCHB, CHB))


def _gemm_kernel(be_ref, xg_ref, w1_ref, b1_ref, w2_ref, b2_ref, yg_ref):
    del be_ref
    xgb = xg_ref[...].astype(jnp.bfloat16)
    h = lax.dot_general(xgb, w1_ref[0], (((1,), (1,)), ((), ())),
                        preferred_element_type=jnp.float32)
    h = jnp.maximum(h + b1_ref[0], 0.0)
    y = lax.dot_general(h.astype(jnp.bfloat16), w2_ref[0],
                        (((1,), (1,)), ((), ())),
                        preferred_element_type=jnp.float32)
    yg_ref[...] = y + b2_ref[0]


def _run_gemm(block_expert, xg, W1b, b1, W2b, b2):
    grid_spec = pltpu.PrefetchScalarGridSpec(
        num_scalar_prefetch=1,
        grid=(NBLK,),
        in_specs=[
            pl.BlockSpec((GB, M), lambda i, be: (i, 0)),
            pl.BlockSpec((1, HIDDEN, M), lambda i, be: (be[i], 0, 0)),
            pl.BlockSpec((1, 1, HIDDEN), lambda i, be: (be[i], 0, 0)),
            pl.BlockSpec((1, M, HIDDEN), lambda i, be: (be[i], 0, 0)),
            pl.BlockSpec((1, 1, M), lambda i, be: (be[i], 0, 0)),
        ],
        out_specs=pl.BlockSpec((GB, M), lambda i, be: (i, 0)),
    )
    return pl.pallas_call(
        _gemm_kernel,
        grid_spec=grid_spec,
        out_shape=jax.ShapeDtypeStruct((P_PAD, M), jnp.float32),
    )(block_expert, xg, W1b, b1, W2b, b2)


CHD = 8
NCHD = TPW // CHD


def _combine_body(yg_hbm, di_hbm, w1_hbm, w2_hbm, out_hbm,
                  di_v, w1r_v, w2r_v, yi_v, o_v, gsem, stsem):
    c = lax.axis_index("c")
    s = lax.axis_index("s")
    wid = s * 2 + c
    base = wid * TPW
    pltpu.sync_copy(di_hbm.at[wid], di_v)
    pltpu.sync_copy(w1_hbm.at[pl.ds(base, TPW)], w1r_v)
    pltpu.sync_copy(w2_hbm.at[pl.ds(base, TPW)], w2r_v)
    gats = [None] * NCHD
    stos = [None] * NCHD
    for j in range(min(2, NCHD)):
        gats[j] = pltpu.async_copy(yg_hbm.at[di_v[j]], yi_v.at[j % 2], gsem)
    for j in range(NCHD):
        b = j % 2
        gats[j].wait()
        if j >= 1:
            stos[j - 1].wait()
        def tok_body(tt, _, b=b, j=j):
            w1s = w1r_v[j * CHD + tt]
            w2s = w2r_v[j * CHD + tt]

            def col_body(q, _):
                cs = q * 16
                o_v[0, tt, pl.ds(cs, 16)] = (
                    w1s * yi_v[b, 2 * tt, pl.ds(cs, 16)]
                    + w2s * yi_v[b, 2 * tt + 1, pl.ds(cs, 16)])
                return 0

            lax.fori_loop(0, M // 16, col_body, 0, unroll=4)
            return 0

        lax.fori_loop(0, CHD, tok_body, 0)
        stos[j] = pltpu.async_copy(
            o_v.at[0], out_hbm.at[pl.ds(base + j * CHD, CHD)], stsem)
        nxt = j + 2
        if nxt < NCHD:
            gats[nxt] = pltpu.async_copy(
                yg_hbm.at[di_v[nxt]], yi_v.at[b], gsem)
    stos[NCHD - 1].wait()


def _run_combine(yg, dint, w1rep, w2rep):
    mesh = plsc.VectorSubcoreMesh(core_axis_name="c", subcore_axis_name="s",
                                   num_cores=2, num_subcores=16)
    fn = pl.kernel(
        _combine_body,
        out_type=jax.ShapeDtypeStruct((N_TOK, M), jnp.float32),
        mesh=mesh,
        compiler_params=pltpu.CompilerParams(needs_layout_passes=False),
        scratch_types=[
            pltpu.VMEM((NCHD, 2 * CHD), jnp.int32),
            pltpu.VMEM((TPW, 16), jnp.float32),
            pltpu.VMEM((TPW, 16), jnp.float32),
            pltpu.VMEM((2, 2 * CHD, M), jnp.float32),
            pltpu.VMEM((1, CHD, M), jnp.float32),
            pltpu.SemaphoreType.DMA,
            pltpu.SemaphoreType.DMA,
        ],
    )
    return fn(yg, dint.reshape(NW, NCHD, 2 * CHD), w1rep, w2rep)


def kernel(x, router_w, W1, b1, W2, b2):
    xb = x.astype(jnp.bfloat16)
    rwb = router_w.astype(jnp.bfloat16)
    W1b = W1.astype(jnp.bfloat16)
    W2b = W2.astype(jnp.bfloat16)

    e1, e2, r1, r2, w1v, w2v, cnt = _run_router(xb, rwb)
    e1 = e1.reshape(N_TOK)
    e2 = e2.reshape(N_TOK)
    r1 = r1.reshape(N_TOK)
    r2 = r2.reshape(N_TOK)
    w1v = w1v.reshape(N_TOK)
    w2v = w2v.reshape(N_TOK)

    counts = cnt[0]
    nb = (counts + (GB - 1)) // GB
    pend = jnp.cumsum(nb * GB)
    poff = (pend - nb * GB).astype(jnp.int32)
    poff = jnp.pad(poff, (0, 16 - NUM_EXPERTS))
    blk_start = jnp.arange(NBLK, dtype=jnp.int32) * GB
    block_expert = jnp.minimum(
        jnp.searchsorted(pend, blk_start, side="right"),
        NUM_EXPERTS - 1).astype(jnp.int32)

    dest1 = poff[e1] + r1
    dest2 = poff[e2] + r2
    xg = _run_dispatch(x, dest1, dest2)
    yg = _run_gemm(block_expert, xg, W1b,
                   b1.reshape(NUM_EXPERTS, 1, HIDDEN), W2b,
                   b2.reshape(NUM_EXPERTS, 1, M))

    dint = jnp.stack([dest1, dest2], axis=1).reshape(2 * N_TOK)
    w1rep = jnp.broadcast_to(w1v[:, None], (N_TOK, 16))
    w2rep = jnp.broadcast_to(w2v[:, None], (N_TOK, 16))
    return _run_combine(yg, dint, w1rep, w2rep)


# speedup vs baseline: 1.2603x; 1.2603x over previous
import jax
import jax.numpy as jnp
from jax import lax
from jax.experimental import pallas as pl
from jax.experimental.pallas import tpu as pltpu
from jax.experimental.pallas import tpu_sc as plsc

N_TOK = 4096
M = 2048
HIDDEN = 512
NUM_EXPERTS = 8
TB = 256
N_TB = N_TOK // TB
GB = 256
CAP = N_TOK
CAPB = CAP // GB
NROW = NUM_EXPERTS * CAP
NBLK = 2 * N_TOK // GB + NUM_EXPERTS
NW = 32
TPW = N_TOK // NW


def _router_kernel(x_ref, rw_ref, tri_ref, d1_ref, d2_ref, di_ref,
                   w1_ref, w2_ref, cnt_ref, carry_ref):
    i = pl.program_id(0)

    @pl.when(i == 0)
    def _():
        carry_ref[...] = jnp.zeros_like(carry_ref)

    xb = x_ref[...]
    logits = lax.dot_general(xb, rw_ref[...], (((1,), (1,)), ((), ())),
                             preferred_element_type=jnp.float32)
    m1 = jnp.max(logits, axis=1, keepdims=True)
    cols = lax.broadcasted_iota(jnp.int32, logits.shape, 1)
    idx1 = jnp.min(jnp.where(logits == m1, cols, NUM_EXPERTS),
                   axis=1, keepdims=True)
    masked = jnp.where(cols == idx1, -jnp.inf, logits)
    m2 = jnp.max(masked, axis=1, keepdims=True)
    idx2 = jnp.min(jnp.where(masked == m2, cols, NUM_EXPERTS),
                   axis=1, keepdims=True)
    ex = jnp.exp(m2 - m1)
    den = 1.0 + ex
    w1v = 1.0 / den
    w2v = ex / den

    oh1 = (cols == idx1).astype(jnp.float32)
    oh2 = (cols == idx2).astype(jnp.float32)
    tri = tri_ref[...]
    pre1 = lax.dot_general(tri, oh1, (((1,), (0,)), ((), ())),
                           preferred_element_type=jnp.float32)
    pre2 = lax.dot_general(tri, oh2, (((1,), (0,)), ((), ())),
                           preferred_element_type=jnp.float32)
    sum1 = jnp.sum(oh1, axis=0, keepdims=True)
    sum2 = jnp.sum(oh2, axis=0, keepdims=True)
    carry = carry_ref[...]
    rank1 = jnp.sum(jnp.where(cols == idx1, pre1 + carry, 0.0),
                    axis=1, keepdims=True)
    rank2 = jnp.sum(jnp.where(cols == idx2, pre2 + sum1 + carry, 0.0),
                    axis=1, keepdims=True)
    new_carry = carry + sum1 + sum2
    carry_ref[...] = new_carry
    cnt_ref[...] = new_carry.astype(jnp.int32)

    dest1 = idx1 * CAP + rank1.astype(jnp.int32)
    dest2 = idx2 * CAP + rank2.astype(jnp.int32)
    d1_ref[...] = dest1.reshape(1, TB, 1)
    d2_ref[...] = dest2.reshape(1, TB, 1)
    di_ref[...] = jnp.concatenate([dest1, dest2], axis=1).reshape(1, TB, 2)
    w1_ref[...] = jnp.broadcast_to(w1v, (TB, 16)).reshape(1, TB, 16)
    w2_ref[...] = jnp.broadcast_to(w2v, (TB, 16)).reshape(1, TB, 16)


def _run_router(x, rw):
    tri = jnp.tril(jnp.ones((TB, TB), jnp.float32), -1)
    vec = jax.ShapeDtypeStruct((N_TB, TB, 1), jnp.int32)
    blk = pl.BlockSpec((1, TB, 1), lambda i: (i, 0, 0))
    return pl.pallas_call(
        _router_kernel,
        grid=(N_TB,),
        in_specs=[
            pl.BlockSpec((TB, M), lambda i: (i, 0)),
            pl.BlockSpec((NUM_EXPERTS, M), lambda i: (0, 0)),
            pl.BlockSpec((TB, TB), lambda i: (0, 0)),
        ],
        out_specs=[blk, blk,
                   pl.BlockSpec((1, TB, 2), lambda i: (i, 0, 0)),
                   pl.BlockSpec((1, TB, 16), lambda i: (i, 0, 0)),
                   pl.BlockSpec((1, TB, 16), lambda i: (i, 0, 0)),
                   pl.BlockSpec((1, NUM_EXPERTS), lambda i: (0, 0))],
        out_shape=[vec, vec,
                   jax.ShapeDtypeStruct((N_TB, TB, 2), jnp.int32),
                   jax.ShapeDtypeStruct((N_TB, TB, 16), jnp.float32),
                   jax.ShapeDtypeStruct((N_TB, TB, 16), jnp.float32),
                   jax.ShapeDtypeStruct((1, NUM_EXPERTS), jnp.int32)],
        scratch_shapes=[pltpu.VMEM((1, NUM_EXPERTS), jnp.float32)],
    )(x, rw, tri)


CHB = 16
NCHB = TPW // CHB


def _dispatch_body(x_hbm, d1_hbm, d2_hbm, xg_hbm,
                   d1_v, d2_v, xrow_v, lsem, ssem):
    c = lax.axis_index("c")
    s = lax.axis_index("s")
    wid = s * 2 + c
    base = wid * TPW
    pltpu.sync_copy(d1_hbm.at[wid], d1_v)
    pltpu.sync_copy(d2_hbm.at[wid], d2_v)
    loads = [None] * NCHB
    scats = [None] * NCHB
    for j in range(min(2, NCHB)):
        loads[j] = pltpu.async_copy(
            x_hbm.at[pl.ds(base + j * CHB, CHB)], xrow_v.at[j % 2], lsem)
    for j in range(NCHB):
        b = j % 2
        loads[j].wait()
        i1 = d1_v[j]
        i2 = d2_v[j]
        scats[j] = (pltpu.async_copy(xrow_v.at[b], xg_hbm.at[i1], ssem),
                    pltpu.async_copy(xrow_v.at[b], xg_hbm.at[i2], ssem))
        nxt = j + 2
        if nxt < NCHB:
            for cd in scats[nxt - 2]:
                cd.wait()
            loads[nxt] = pltpu.async_copy(
                x_hbm.at[pl.ds(base + nxt * CHB, CHB)], xrow_v.at[b], lsem)
    for j in range(max(0, NCHB - 2), NCHB):
        for cd in scats[j]:
            cd.wait()


def _run_dispatch(x, d1, d2):
    mesh = plsc.VectorSubcoreMesh(core_axis_name="c", subcore_axis_name="s",
                                  num_cores=2, num_subcores=16)
    fn = pl.kernel(
        _dispatch_body,
        out_type=jax.ShapeDtypeStruct((NROW, M), jnp.float32),
        mesh=mesh,
        compiler_params=pltpu.CompilerParams(needs_layout_passes=False),
        scratch_types=[
            pltpu.VMEM((NCHB, CHB), jnp.int32),
            pltpu.VMEM((NCHB, CHB), jnp.int32),
            pltpu.VMEM((2, CHB, M), jnp.float32),
            pltpu.SemaphoreType.DMA,
            pltpu.SemaphoreType.DMA,
        ],
    )
    return fn(x, d1.reshape(NW, NCHB, CHB), d2.reshape(NW, NCHB, CHB))


def _gemm_kernel(be_ref, rm_ref, xg_ref, w1_ref, b1_ref, w2_ref, b2_ref,
                 yg_ref):
    del be_ref, rm_ref
    h = lax.dot_general(xg_ref[...], w1_ref[0], (((1,), (1,)), ((), ())),
                        preferred_element_type=jnp.float32)
    h = jnp.maximum(h + b1_ref[0], 0.0)
    y = lax.dot_general(h, w2_ref[0], (((1,), (1,)), ((), ())),
                        preferred_element_type=jnp.float32)
    yg_ref[...] = y + b2_ref[0]


def _run_gemm(be, rowmap, xg, W1, b1, W2, b2):
    grid_spec = pltpu.PrefetchScalarGridSpec(
        num_scalar_prefetch=2,
        grid=(NBLK,),
        in_specs=[
            pl.BlockSpec((GB, M), lambda i, be, rm: (rm[i], 0)),
            pl.BlockSpec((1, HIDDEN, M), lambda i, be, rm: (be[i], 0, 0)),
            pl.BlockSpec((1, 1, HIDDEN), lambda i, be, rm: (be[i], 0, 0)),
            pl.BlockSpec((1, M, HIDDEN), lambda i, be, rm: (be[i], 0, 0)),
            pl.BlockSpec((1, 1, M), lambda i, be, rm: (be[i], 0, 0)),
        ],
        out_specs=pl.BlockSpec((GB, M), lambda i, be, rm: (rm[i], 0)),
    )
    return pl.pallas_call(
        _gemm_kernel,
        grid_spec=grid_spec,
        out_shape=jax.ShapeDtypeStruct((NROW, M), jnp.float32),
    )(be, rowmap, xg, W1, b1.reshape(NUM_EXPERTS, 1, HIDDEN),
      W2, b2.reshape(NUM_EXPERTS, 1, M))


CHD = 8
NCHD = TPW // CHD


def _combine_body(yg_hbm, di_hbm, w1_hbm, w2_hbm, out_hbm,
                  di_v, w1r_v, w2r_v, yi_v, o_v, gsem, stsem):
    c = lax.axis_index("c")
    s = lax.axis_index("s")
    wid = s * 2 + c
    base = wid * TPW
    pltpu.sync_copy(di_hbm.at[wid], di_v)
    pltpu.sync_copy(w1_hbm.at[pl.ds(base, TPW)], w1r_v)
    pltpu.sync_copy(w2_hbm.at[pl.ds(base, TPW)], w2r_v)
    gats = [None] * NCHD
    stos = [None] * NCHD
    for j in range(min(2, NCHD)):
        gats[j] = pltpu.async_copy(yg_hbm.at[di_v[j]], yi_v.at[j % 2], gsem)
    for j in range(NCHD):
        b = j % 2
        gats[j].wait()
        if j >= 1:
            stos[j - 1].wait()

        def tok_body(tt, _, b=b, j=j):
            w1s = w1r_v[j * CHD + tt]
            w2s = w2r_v[j * CHD + tt]

            def col_body(q, _):
                cs = q * 16
                o_v[0, tt, pl.ds(cs, 16)] = (
                    w1s * yi_v[b, 2 * tt, pl.ds(cs, 16)]
                    + w2s * yi_v[b, 2 * tt + 1, pl.ds(cs, 16)])
                return 0

            lax.fori_loop(0, M // 16, col_body, 0, unroll=4)
            return 0

        lax.fori_loop(0, CHD, tok_body, 0)
        stos[j] = pltpu.async_copy(
            o_v.at[0], out_hbm.at[pl.ds(base + j * CHD, CHD)], stsem)
        nxt = j + 2
        if nxt < NCHD:
            gats[nxt] = pltpu.async_copy(
                yg_hbm.at[di_v[nxt]], yi_v.at[b], gsem)
    stos[NCHD - 1].wait()


def _run_combine(yg, dint, w1rep, w2rep):
    mesh = plsc.VectorSubcoreMesh(core_axis_name="c", subcore_axis_name="s",
                                  num_cores=2, num_subcores=16)
    fn = pl.kernel(
        _combine_body,
        out_type=jax.ShapeDtypeStruct((N_TOK, M), jnp.float32),
        mesh=mesh,
        compiler_params=pltpu.CompilerParams(needs_layout_passes=False),
        scratch_types=[
            pltpu.VMEM((NCHD, 2 * CHD), jnp.int32),
            pltpu.VMEM((TPW, 16), jnp.float32),
            pltpu.VMEM((TPW, 16), jnp.float32),
            pltpu.VMEM((2, 2 * CHD, M), jnp.float32),
            pltpu.VMEM((1, CHD, M), jnp.float32),
            pltpu.SemaphoreType.DMA,
            pltpu.SemaphoreType.DMA,
        ],
    )
    return fn(yg, dint.reshape(NW, NCHD, 2 * CHD), w1rep, w2rep)


def kernel(x, router_w, W1, b1, W2, b2):
    d1, d2, dint, w1rep, w2rep, cnt = _run_router(x, router_w)
    d1 = d1.reshape(N_TOK)
    d2 = d2.reshape(N_TOK)
    dint = dint.reshape(2 * N_TOK)
    w1rep = w1rep.reshape(N_TOK, 16)
    w2rep = w2rep.reshape(N_TOK, 16)

    counts = cnt[0]
    nb = (counts + (GB - 1)) // GB
    pend = jnp.cumsum(nb)
    pstart = pend - nb
    j = jnp.arange(NBLK, dtype=jnp.int32)
    be = jnp.minimum(jnp.sum((pend[None, :] <= j[:, None]).astype(jnp.int32),
                             axis=1), NUM_EXPERTS - 1).astype(jnp.int32)
    bs = jnp.clip(j - pstart[be], 0, CAPB - 1).astype(jnp.int32)
    rowmap = (be * CAPB + bs).astype(jnp.int32)

    xg = _run_dispatch(x, d1, d2)
    yg = _run_gemm(be, rowmap, xg, W1, b1, W2, b2)
    return _run_combine(yg, dint, w1rep, w2rep)
